# Initial kernel scaffold; baseline (speedup 1.0000x reference)
import jax
import jax.numpy as jnp
from jax.experimental import pallas as pl


def kernel(x, word_table, pos_table):
    raise NotImplementedError("write your pallas kernel here")


# speedup vs baseline: 2.4543x; 2.4543x over previous
import jax
import jax.numpy as jnp
from jax import lax
from jax.experimental import pallas as pl
from jax.experimental.pallas import tpu as pltpu
from jax.experimental.pallas import tpu_sc as plsc

NC, NS, L = 2, 16, 16
NW = NC * NS
BATCH, SEQ, DIM = 4096, 200, 64
RPW = BATCH // NW
C0, C1 = 104, 96


def _body(x_hbm, wt_hbm, pt_hbm, out_hbm, idx_v, pos_v, buf, gsem):
    wid = lax.axis_index("s") * NC + lax.axis_index("c")
    base = wid * RPW
    pltpu.sync_copy(pt_hbm, pos_v)
    pltpu.sync_copy(x_hbm.at[pl.ds(base, RPW)], idx_v)

    @pl.loop(0, RPW)
    def _row(r):
        d0 = pltpu.async_copy(
            wt_hbm.at[idx_v.at[r, pl.ds(0, C0)]], buf.at[pl.ds(0, C0)], gsem)
        d1 = pltpu.async_copy(
            wt_hbm.at[idx_v.at[r, pl.ds(C0, C1)]], buf.at[pl.ds(C0, C1)], gsem)
        d0.wait()
        d1.wait()

        @pl.loop(0, SEQ)
        def _seq(s):
            for c in range(DIM // L):
                plsc.addupdate(buf.at[s, pl.ds(c * L, L)],
                               pos_v[s, pl.ds(c * L, L)])

        pltpu.sync_copy(buf, out_hbm.at[base + r])


def kernel(x, word_table, pos_table):
    x = x.astype(jnp.int32)
    mesh = plsc.VectorSubcoreMesh(core_axis_name="c", subcore_axis_name="s")
    f = pl.kernel(
        _body,
        out_type=jax.ShapeDtypeStruct((BATCH, SEQ, DIM), jnp.float32),
        mesh=mesh,
        scratch_types=[
            pltpu.VMEM((RPW, SEQ), jnp.int32),
            pltpu.VMEM((SEQ, DIM), jnp.float32),
            pltpu.VMEM((SEQ, DIM), jnp.float32),
            pltpu.SemaphoreType.DMA,
        ],
        compiler_params=pltpu.CompilerParams(use_tc_tiling_on_sc=False),
    )
    return f(x, word_table, pos_table)


# speedup vs baseline: 2.7323x; 1.1133x over previous
import jax
import jax.numpy as jnp
from jax import lax
from jax.experimental import pallas as pl
from jax.experimental.pallas import tpu as pltpu
from jax.experimental.pallas import tpu_sc as plsc

NC, NS, L = 2, 16, 16
NW = NC * NS
BATCH, SEQ, DIM = 4096, 200, 64
RPW = BATCH // NW
C0, C1 = 104, 96
NBUF = 4


def _body(x_hbm, wt_hbm, pt_hbm, out_hbm, idx_v, pos_v, buf, gsem, osem):
    wid = lax.axis_index("s") * NC + lax.axis_index("c")
    base = wid * RPW
    pltpu.sync_copy(pt_hbm, pos_v)
    pltpu.sync_copy(x_hbm.at[pl.ds(base, RPW)], idx_v)

    def fire_gather(j, r):
        pltpu.async_copy(wt_hbm.at[idx_v.at[r, pl.ds(0, C0)]],
                         buf.at[j, pl.ds(0, C0)], gsem.at[j])
        pltpu.async_copy(wt_hbm.at[idx_v.at[r, pl.ds(C0, C1)]],
                         buf.at[j, pl.ds(C0, C1)], gsem.at[j])

    def wait_gather(j):
        pltpu.make_async_copy(wt_hbm.at[pl.ds(0, SEQ)], buf.at[j],
                              gsem.at[j]).wait()

    def pos_add(j):
        @pl.loop(0, SEQ, unroll=8)
        def _seq(s):
            for c in range(DIM // L):
                plsc.addupdate(buf.at[j, s, pl.ds(c * L, L)],
                               pos_v[s, pl.ds(c * L, L)])

    def fire_out(j, r):
        pltpu.async_copy(buf.at[j], out_hbm.at[base + r], osem.at[j])

    def wait_out(j):
        pltpu.make_async_copy(buf.at[j], out_hbm.at[0], osem.at[j]).wait()

    for j in range(NBUF):
        fire_gather(j, j)

    @pl.loop(0, RPW - NBUF, step=NBUF)
    def _ring(g):
        for j in range(NBUF):
            wait_gather(j)
            pos_add(j)
            fire_out(j, g + j)
        for j in range(NBUF):
            wait_out(j)
            fire_gather(j, g + NBUF + j)

    for j in range(NBUF):
        wait_gather(j)
        pos_add(j)
        fire_out(j, RPW - NBUF + j)
    for j in range(NBUF):
        wait_out(j)


def kernel(x, word_table, pos_table):
    x = x.astype(jnp.int32)
    mesh = plsc.VectorSubcoreMesh(core_axis_name="c", subcore_axis_name="s")
    f = pl.kernel(
        _body,
        out_type=jax.ShapeDtypeStruct((BATCH, SEQ, DIM), jnp.float32),
        mesh=mesh,
        scratch_types=[
            pltpu.VMEM((RPW, SEQ), jnp.int32),
            pltpu.VMEM((SEQ, DIM), jnp.float32),
            pltpu.VMEM((NBUF, SEQ, DIM), jnp.float32),
            pltpu.SemaphoreType.DMA((NBUF,)),
            pltpu.SemaphoreType.DMA((NBUF,)),
        ],
        compiler_params=pltpu.CompilerParams(use_tc_tiling_on_sc=False),
    )
    return f(x, word_table, pos_table)
